# baseline (speedup 1.0000x reference)
import functools

import jax
import jax.numpy as jnp
from jax import lax
from jax.experimental import pallas as pl
from jax.experimental.pallas import tpu as pltpu
from jax.experimental.pallas import tpu_sc as plsc

CORE0_FRAC = 0.425

NC = 2
NS = 16
NW = NC * NS
LANES = 16


def _round_up(a, b):
    return (a + b - 1) // b * b


G = 128


def _pick_stage_rows(cpt):
    for kb in (32, 24, 16, 8):
        if cpt % kb == 0:
            return kb
    raise ValueError(f"chunk count {cpt} not a multiple of 8")


def _sc_degree(dst2, w2, n):
    nchunk, g = dst2.shape
    npad = _round_up(n, 8 * NW)
    cpt = nchunk // NW
    rps = npad // NS
    kb = _pick_stage_rows(cpt)
    mesh = plsc.VectorSubcoreMesh(core_axis_name="c", subcore_axis_name="s")

    @functools.partial(
        pl.kernel,
        out_type=jax.ShapeDtypeStruct((NC, npad), jnp.float32),
        mesh=mesh,
        scratch_types=[
            pltpu.VMEM((kb, g), jnp.int32),
            pltpu.VMEM((kb, g), jnp.float32),
            pltpu.VMEM((rps,), jnp.float32),
            pltpu.VMEM_SHARED((npad,), jnp.float32),
        ],
        compiler_params=pltpu.CompilerParams(use_tc_tiling_on_sc=False),
    )
    def k(dst_hbm, w_hbm, out_hbm, dst_v, w_v, zero_v, deg_sh):
        c = lax.axis_index("c")
        s = lax.axis_index("s")
        wid = c * NS + s

        @pl.loop(0, rps // LANES)
        def _(i):
            zero_v[pl.ds(i * LANES, LANES)] = jnp.zeros((LANES,), jnp.float32)

        pltpu.sync_copy(zero_v, deg_sh.at[pl.ds(s * rps, rps)])
        plsc.subcore_barrier()

        @pl.loop(0, cpt // kb)
        def _(b):
            row0 = wid * cpt + b * kb
            pltpu.sync_copy(dst_hbm.at[pl.ds(row0, kb)], dst_v)
            pltpu.sync_copy(w_hbm.at[pl.ds(row0, kb)], w_v)

            @pl.loop(0, kb)
            def _(j):
                pltpu.sync_copy(w_v.at[j], deg_sh.at[dst_v.at[j]], add=True)

        plsc.subcore_barrier()
        pltpu.sync_copy(deg_sh.at[pl.ds(s * rps, rps)],
                        out_hbm.at[c, pl.ds(s * rps, rps)])

    return k(dst2, w2)


def _sc_edge_pass(hh_bf, src2, dst2, w2):
    npad, d = hh_bf.shape
    nchunk, g = src2.shape
    cpt = nchunk // NW
    rps = npad // NS
    mesh = plsc.VectorSubcoreMesh(core_axis_name="c", subcore_axis_name="s")
    assert cpt % 2 == 0
    zr = 32 if d <= 64 else 16
    assert rps % zr == 0
    f0 = max(4, int(round(cpt * 2 * CORE0_FRAC / 4)) * 4)
    f1 = 2 * cpt - f0
    assert f0 % 4 == 0 and f1 % 4 == 0 and f0 >= 8 and f1 >= 8
    cmax = max(f0, f1)

    @functools.partial(
        pl.kernel,
        out_type=jax.ShapeDtypeStruct((NC, npad, d), jnp.float32),
        mesh=mesh,
        scratch_types=[
            pltpu.VMEM((cmax, g), jnp.int32),
            pltpu.VMEM((cmax, g), jnp.int32),
            pltpu.VMEM((cmax, g), jnp.float32),
            pltpu.VMEM((g, d), jnp.bfloat16),
            pltpu.VMEM((g, d), jnp.bfloat16),
            pltpu.VMEM((g, d), jnp.bfloat16),
            pltpu.VMEM((g, d), jnp.bfloat16),
            pltpu.VMEM((g, d), jnp.float32),
            pltpu.VMEM((g, d), jnp.float32),
            pltpu.VMEM((g, d), jnp.float32),
            pltpu.VMEM((g, d), jnp.float32),
            pltpu.VMEM((zr, d), jnp.float32),
            pltpu.VMEM_SHARED((npad, d), jnp.float32),
            pltpu.SemaphoreType.DMA,
            pltpu.SemaphoreType.DMA,
            pltpu.SemaphoreType.DMA,
            pltpu.SemaphoreType.DMA,
            pltpu.SemaphoreType.DMA,
            pltpu.SemaphoreType.DMA,
            pltpu.SemaphoreType.DMA,
            pltpu.SemaphoreType.DMA,
            pltpu.SemaphoreType.DMA,
            pltpu.SemaphoreType.DMA,
        ],
        compiler_params=pltpu.CompilerParams(
            use_tc_tiling_on_sc=False, needs_layout_passes=False),
    )
    def k(hh_hbm, src_hbm, dst_hbm, w_hbm, out_hbm,
          src_v, dst_v, w_v, in0, in1, in2, in3, out0, out1, out2, out3,
          zero_v, acc_sh, isem, g0, g1, g2, g3, s0, s1, s2, s3, zsem):
        c = lax.axis_index("c")
        s = lax.axis_index("s")
        cpt_c = jnp.where(c == 0, f0, f1)
        row0 = c * (NS * f0) + s * cpt_c
        ibufs = (in0, in1, in2, in3)
        obufs = (out0, out1, out2, out3)
        gsems = (g0, g1, g2, g3)
        ssems = (s0, s1, s2, s3)

        i1 = pltpu.async_copy(src_hbm.at[pl.ds(row0, cmax)], src_v, isem)
        i2 = pltpu.async_copy(dst_hbm.at[pl.ds(row0, cmax)], dst_v, isem)
        i3 = pltpu.async_copy(w_hbm.at[pl.ds(row0, cmax)], w_v, isem)

        @pl.loop(0, zr)
        def _(r):
            for j in range(d // LANES):
                zero_v[r, pl.ds(j * LANES, LANES)] = jnp.zeros((LANES,), jnp.float32)

        for i in range(rps // zr):
            pltpu.async_copy(zero_v, acc_sh.at[pl.ds(s * rps + i * zr, zr)],
                             zsem)
        for i in range(rps // zr):
            pltpu.make_async_copy(
                zero_v, acc_sh.at[pl.ds(s * rps + i * zr, zr)], zsem).wait()

        i1.wait()
        i2.wait()
        i3.wait()
        plsc.subcore_barrier()

        def scale(jj, ibuf, obuf):
            @pl.loop(0, g // LANES)
            def _(eb):
                wvec = w_v[jj, pl.ds(eb * LANES, LANES)]
                for t in range(LANES):
                    wb = lax.broadcast(wvec[t], (LANES,))
                    e = eb * LANES + t
                    for q in range(d // 32):
                        x = ibuf[e, pl.ds(q * 32, 32)]
                        a, b = plsc.unpack(x, format=plsc.PackFormat.INTERLEAVED)
                        obuf[e, pl.ds(q * 32, LANES)] = a * wb
                        obuf[e, pl.ds(q * 32 + LANES, LANES)] = b * wb

        pltpu.async_copy(hh_hbm.at[src_v.at[0]], in0, g0)
        pltpu.async_copy(hh_hbm.at[src_v.at[1]], in1, g1)
        pltpu.async_copy(hh_hbm.at[src_v.at[2]], in2, g2)

        @pl.loop(0, cpt_c // 4)
        def _(q):
            for l in range(4):
                jj = q * 4 + l
                ibuf, obuf, gs, ss = ibufs[l], obufs[l], gsems[l], ssems[l]
                nin, ngs = ibufs[(l + 3) % 4], gsems[(l + 3) % 4]
                pout, pss = obufs[(l + 1) % 4], ssems[(l + 1) % 4]
                pltpu.make_async_copy(hh_hbm.at[src_v.at[jj]], ibuf, gs).wait()

                @pl.when(jj + 3 < cpt_c)
                def _():
                    pltpu.async_copy(hh_hbm.at[src_v.at[jj + 3]], nin, ngs)

                @pl.when(jj >= 3)
                def _():
                    pltpu.make_async_copy(
                        pout, acc_sh.at[dst_v.at[jj]], pss).wait()

                scale(jj, ibuf, obuf)
                pltpu.async_copy(obuf, acc_sh.at[dst_v.at[jj]], ss, add=True)

        for t in range(3):
            pltpu.make_async_copy(out0, acc_sh.at[dst_v.at[0]],
                                  ssems[1 + t]).wait()

        plsc.subcore_barrier()
        pltpu.sync_copy(acc_sh.at[pl.ds(s * rps, rps)],
                        out_hbm.at[c, pl.ds(s * rps, rps)])

    return k(hh_bf, src2, dst2, w2)


def _tc_matmul(a, w):
    n, din = a.shape
    dout = w.shape[1]
    bn = 2000

    def body(a_ref, w_ref, o_ref):
        o_ref[...] = jnp.dot(a_ref[...], w_ref[...],
                             preferred_element_type=jnp.float32)

    return pl.pallas_call(
        body,
        grid=(n // bn,),
        in_specs=[pl.BlockSpec((bn, din), lambda i: (i, 0)),
                  pl.BlockSpec((din, dout), lambda i: (0, 0))],
        out_specs=pl.BlockSpec((bn, dout), lambda i: (i, 0)),
        out_shape=jax.ShapeDtypeStruct((n, dout), jnp.float32),
    )(a, w)


def _tc_dinv_hh(degt, h):
    n, d = h.shape
    bn = 2000

    def body(deg_ref, h_ref, dinv_ref, hh_ref):
        deg = jnp.sum(deg_ref[...], axis=1, keepdims=True) + 1.0
        dinv = jnp.where(deg > 0, lax.rsqrt(deg), 0.0)
        dinv_ref[...] = dinv
        hh_ref[...] = dinv * h_ref[...]

    return pl.pallas_call(
        body,
        grid=(n // bn,),
        in_specs=[pl.BlockSpec((bn, NC), lambda i: (i, 0)),
                  pl.BlockSpec((bn, d), lambda i: (i, 0))],
        out_specs=[pl.BlockSpec((bn, 1), lambda i: (i, 0)),
                   pl.BlockSpec((bn, d), lambda i: (i, 0))],
        out_shape=[jax.ShapeDtypeStruct((n, 1), jnp.float32),
                   jax.ShapeDtypeStruct((n, d), jnp.float32)],
    )(degt, h)


def _tc_mid(parts, hh, dinv, b, w):
    n, d = hh.shape
    dout = w.shape[1]
    bn = 2000

    def body(p_ref, hh_ref, dinv_ref, b_ref, w_ref, o_ref):
        acc = p_ref[0] + p_ref[1] + hh_ref[...]
        act = jnp.maximum(dinv_ref[...] * acc + b_ref[...], 0.0)
        o_ref[...] = dinv_ref[...] * jnp.dot(
            act, w_ref[...], preferred_element_type=jnp.float32)

    return pl.pallas_call(
        body,
        grid=(n // bn,),
        in_specs=[pl.BlockSpec((NC, bn, d), lambda i: (0, i, 0)),
                  pl.BlockSpec((bn, d), lambda i: (i, 0)),
                  pl.BlockSpec((bn, 1), lambda i: (i, 0)),
                  pl.BlockSpec((1, d), lambda i: (0, 0)),
                  pl.BlockSpec((d, dout), lambda i: (0, 0))],
        out_specs=pl.BlockSpec((bn, dout), lambda i: (i, 0)),
        out_shape=jax.ShapeDtypeStruct((n, dout), jnp.float32),
    )(parts, hh, dinv, b, w)


def _tc_mid2(pa, pb, hh, dinv, b, w):
    n, d = hh.shape
    dh = d // 2
    dout = w.shape[1]
    bn = 2000

    def body(pa_ref, pb_ref, hh_ref, dinv_ref, b_ref, w_ref, o_ref):
        acc = jnp.concatenate(
            [pa_ref[0] + pa_ref[1], pb_ref[0] + pb_ref[1]], axis=1)
        acc = acc + hh_ref[...]
        act = jnp.maximum(dinv_ref[...] * acc + b_ref[...], 0.0)
        o_ref[...] = dinv_ref[...] * jnp.dot(
            act, w_ref[...], preferred_element_type=jnp.float32)

    return pl.pallas_call(
        body,
        grid=(n // bn,),
        in_specs=[pl.BlockSpec((NC, bn, dh), lambda i: (0, i, 0)),
                  pl.BlockSpec((NC, bn, dh), lambda i: (0, i, 0)),
                  pl.BlockSpec((bn, d), lambda i: (i, 0)),
                  pl.BlockSpec((bn, 1), lambda i: (i, 0)),
                  pl.BlockSpec((1, d), lambda i: (0, 0)),
                  pl.BlockSpec((d, dout), lambda i: (0, 0))],
        out_specs=pl.BlockSpec((bn, dout), lambda i: (i, 0)),
        out_shape=jax.ShapeDtypeStruct((n, dout), jnp.float32),
    )(pa, pb, hh, dinv, b, w)


def _tc_post(parts, hh, dinv, b):
    n, d = hh.shape
    bn = 2000

    def body(p_ref, hh_ref, dinv_ref, b_ref, o_ref):
        acc = p_ref[0] + p_ref[1] + hh_ref[...]
        o_ref[...] = dinv_ref[...] * acc + b_ref[...]

    return pl.pallas_call(
        body,
        grid=(n // bn,),
        in_specs=[pl.BlockSpec((NC, bn, d), lambda i: (0, i, 0)),
                  pl.BlockSpec((bn, d), lambda i: (i, 0)),
                  pl.BlockSpec((bn, 1), lambda i: (i, 0)),
                  pl.BlockSpec((1, d), lambda i: (0, 0))],
        out_specs=pl.BlockSpec((bn, d), lambda i: (i, 0)),
        out_shape=jax.ShapeDtypeStruct((n, d), jnp.float32),
    )(parts, hh, dinv, b)


def kernel(x, edge_index, edge_attr, W1, b1, W2, b2, W3, b3):
    n = x.shape[0]
    e = edge_attr.shape[0]
    ept = _round_up(-(-e // NW), G * 8)
    epad = ept * NW
    pad = epad - e
    npad = _round_up(n, 8 * NW)
    padrow = n if npad > n else n - 1
    srcp = jnp.concatenate([edge_index[0], jnp.zeros((pad,), jnp.int32)])
    dstp = jnp.concatenate([edge_index[1], jnp.full((pad,), padrow, jnp.int32)])
    wpad = jnp.concatenate([edge_attr, jnp.zeros((pad,), jnp.float32)])
    src2 = srcp.reshape(epad // G, G)
    dst2 = dstp.reshape(epad // G, G)
    w2 = wpad.reshape(epad // G, G)
    gw = 32
    src2b = srcp.reshape(epad // gw, gw)
    dst2b = dstp.reshape(epad // gw, gw)
    w2b = wpad.reshape(epad // gw, gw)

    degp = _sc_degree(dst2, w2, n)
    h1 = _tc_matmul(x, W1)
    degt = jnp.transpose(degp)[:n]
    dinv, hh1 = _tc_dinv_hh(degt, h1)

    def prep(a):
        m, d = a.shape
        a = a.astype(jnp.bfloat16)
        a = a.reshape(m, d // 32, 2, 16).transpose(0, 1, 3, 2).reshape(m, d)
        return jnp.pad(a, ((0, npad - m), (0, 0)))

    dh = hh1.shape[1] // 2
    p1a = _sc_edge_pass(prep(hh1[:, :dh]), src2, dst2, w2)
    p1b = _sc_edge_pass(prep(hh1[:, dh:]), src2, dst2, w2)
    hh2 = _tc_mid2(p1a, p1b, hh1, dinv, b1.reshape(1, -1), W2)
    p2 = _sc_edge_pass(prep(hh2), src2, dst2, w2)
    hh3 = _tc_mid(p2, hh2, dinv, b2.reshape(1, -1), W3)
    p3 = _sc_edge_pass(prep(hh3), src2, dst2, w2)
    return _tc_post(p3, hh3, dinv, b3.reshape(1, -1))


# speedup vs baseline: 1.1141x; 1.1141x over previous
import functools

import jax
import jax.numpy as jnp
from jax import lax
from jax.experimental import pallas as pl
from jax.experimental.pallas import tpu as pltpu
from jax.experimental.pallas import tpu_sc as plsc

CORE0_FRAC = 0.575

NC = 2
NS = 16
NW = NC * NS
LANES = 16


def _round_up(a, b):
    return (a + b - 1) // b * b


G = 128


def _pick_stage_rows(cpt):
    for kb in (32, 24, 16, 8):
        if cpt % kb == 0:
            return kb
    raise ValueError(f"chunk count {cpt} not a multiple of 8")


def _sc_degree(dst2, w2, n):
    nchunk, g = dst2.shape
    npad = _round_up(n, 8 * NW)
    cpt = nchunk // NW
    rps = npad // NS
    kb = _pick_stage_rows(cpt)
    mesh = plsc.VectorSubcoreMesh(core_axis_name="c", subcore_axis_name="s")

    @functools.partial(
        pl.kernel,
        out_type=jax.ShapeDtypeStruct((NC, npad), jnp.float32),
        mesh=mesh,
        scratch_types=[
            pltpu.VMEM((kb, g), jnp.int32),
            pltpu.VMEM((kb, g), jnp.float32),
            pltpu.VMEM((rps,), jnp.float32),
            pltpu.VMEM_SHARED((npad,), jnp.float32),
        ],
        compiler_params=pltpu.CompilerParams(use_tc_tiling_on_sc=False),
    )
    def k(dst_hbm, w_hbm, out_hbm, dst_v, w_v, zero_v, deg_sh):
        c = lax.axis_index("c")
        s = lax.axis_index("s")
        wid = c * NS + s

        @pl.loop(0, rps // LANES)
        def _(i):
            zero_v[pl.ds(i * LANES, LANES)] = jnp.zeros((LANES,), jnp.float32)

        pltpu.sync_copy(zero_v, deg_sh.at[pl.ds(s * rps, rps)])
        plsc.subcore_barrier()

        @pl.loop(0, cpt // kb)
        def _(b):
            row0 = wid * cpt + b * kb
            pltpu.sync_copy(dst_hbm.at[pl.ds(row0, kb)], dst_v)
            pltpu.sync_copy(w_hbm.at[pl.ds(row0, kb)], w_v)

            @pl.loop(0, kb)
            def _(j):
                pltpu.sync_copy(w_v.at[j], deg_sh.at[dst_v.at[j]], add=True)

        plsc.subcore_barrier()
        pltpu.sync_copy(deg_sh.at[pl.ds(s * rps, rps)],
                        out_hbm.at[c, pl.ds(s * rps, rps)])

    return k(dst2, w2)


def _sc_edge_pass(hh_bf, src2, dst2, w2):
    npad, d = hh_bf.shape
    nchunk, g = src2.shape
    cpt = nchunk // NW
    rps = npad // NS
    mesh = plsc.VectorSubcoreMesh(core_axis_name="c", subcore_axis_name="s")
    assert cpt % 2 == 0
    zr = 32 if d <= 64 else 16
    assert rps % zr == 0
    f0 = max(4, int(round(cpt * 2 * CORE0_FRAC / 4)) * 4)
    f1 = 2 * cpt - f0
    assert f0 % 4 == 0 and f1 % 4 == 0 and f0 >= 8 and f1 >= 8
    cmax = max(f0, f1)

    @functools.partial(
        pl.kernel,
        out_type=jax.ShapeDtypeStruct((NC, npad, d), jnp.float32),
        mesh=mesh,
        scratch_types=[
            pltpu.VMEM((cmax, g), jnp.int32),
            pltpu.VMEM((cmax, g), jnp.int32),
            pltpu.VMEM((cmax, g), jnp.float32),
            pltpu.VMEM((g, d), jnp.bfloat16),
            pltpu.VMEM((g, d), jnp.bfloat16),
            pltpu.VMEM((g, d), jnp.bfloat16),
            pltpu.VMEM((g, d), jnp.bfloat16),
            pltpu.VMEM((g, d), jnp.float32),
            pltpu.VMEM((g, d), jnp.float32),
            pltpu.VMEM((g, d), jnp.float32),
            pltpu.VMEM((g, d), jnp.float32),
            pltpu.VMEM((zr, d), jnp.float32),
            pltpu.VMEM_SHARED((npad, d), jnp.float32),
            pltpu.SemaphoreType.DMA,
            pltpu.SemaphoreType.DMA,
            pltpu.SemaphoreType.DMA,
            pltpu.SemaphoreType.DMA,
            pltpu.SemaphoreType.DMA,
            pltpu.SemaphoreType.DMA,
            pltpu.SemaphoreType.DMA,
            pltpu.SemaphoreType.DMA,
            pltpu.SemaphoreType.DMA,
            pltpu.SemaphoreType.DMA,
        ],
        compiler_params=pltpu.CompilerParams(
            use_tc_tiling_on_sc=False, needs_layout_passes=False),
    )
    def k(hh_hbm, src_hbm, dst_hbm, w_hbm, out_hbm,
          src_v, dst_v, w_v, in0, in1, in2, in3, out0, out1, out2, out3,
          zero_v, acc_sh, isem, g0, g1, g2, g3, s0, s1, s2, s3, zsem):
        c = lax.axis_index("c")
        s = lax.axis_index("s")
        cpt_c = jnp.where(c == 0, f0, f1)
        row0 = c * (NS * f0) + s * cpt_c
        ibufs = (in0, in1, in2, in3)
        obufs = (out0, out1, out2, out3)
        gsems = (g0, g1, g2, g3)
        ssems = (s0, s1, s2, s3)

        i1 = pltpu.async_copy(src_hbm.at[pl.ds(row0, cmax)], src_v, isem)
        i2 = pltpu.async_copy(dst_hbm.at[pl.ds(row0, cmax)], dst_v, isem)
        i3 = pltpu.async_copy(w_hbm.at[pl.ds(row0, cmax)], w_v, isem)

        @pl.loop(0, zr)
        def _(r):
            for j in range(d // LANES):
                zero_v[r, pl.ds(j * LANES, LANES)] = jnp.zeros((LANES,), jnp.float32)

        for i in range(rps // zr):
            pltpu.async_copy(zero_v, acc_sh.at[pl.ds(s * rps + i * zr, zr)],
                             zsem)
        for i in range(rps // zr):
            pltpu.make_async_copy(
                zero_v, acc_sh.at[pl.ds(s * rps + i * zr, zr)], zsem).wait()

        i1.wait()
        i2.wait()
        i3.wait()
        plsc.subcore_barrier()

        def scale(jj, ibuf, obuf):
            @pl.loop(0, g // LANES)
            def _(eb):
                wvec = w_v[jj, pl.ds(eb * LANES, LANES)]
                for t in range(LANES):
                    wb = lax.broadcast(wvec[t], (LANES,))
                    e = eb * LANES + t
                    for q in range(d // 32):
                        x = ibuf[e, pl.ds(q * 32, 32)]
                        a, b = plsc.unpack(x, format=plsc.PackFormat.INTERLEAVED)
                        obuf[e, pl.ds(q * 32, LANES)] = a * wb
                        obuf[e, pl.ds(q * 32 + LANES, LANES)] = b * wb

        pltpu.async_copy(hh_hbm.at[src_v.at[0]], in0, g0)
        pltpu.async_copy(hh_hbm.at[src_v.at[1]], in1, g1)
        pltpu.async_copy(hh_hbm.at[src_v.at[2]], in2, g2)

        @pl.loop(0, cpt_c // 4)
        def _(q):
            for l in range(4):
                jj = q * 4 + l
                ibuf, obuf, gs, ss = ibufs[l], obufs[l], gsems[l], ssems[l]
                nin, ngs = ibufs[(l + 3) % 4], gsems[(l + 3) % 4]
                pout, pss = obufs[(l + 1) % 4], ssems[(l + 1) % 4]
                pltpu.make_async_copy(hh_hbm.at[src_v.at[jj]], ibuf, gs).wait()

                @pl.when(jj + 3 < cpt_c)
                def _():
                    pltpu.async_copy(hh_hbm.at[src_v.at[jj + 3]], nin, ngs)

                @pl.when(jj >= 3)
                def _():
                    pltpu.make_async_copy(
                        pout, acc_sh.at[dst_v.at[jj]], pss).wait()

                scale(jj, ibuf, obuf)
                pltpu.async_copy(obuf, acc_sh.at[dst_v.at[jj]], ss, add=True)

        for t in range(3):
            pltpu.make_async_copy(out0, acc_sh.at[dst_v.at[0]],
                                  ssems[1 + t]).wait()

        plsc.subcore_barrier()
        pltpu.sync_copy(acc_sh.at[pl.ds(s * rps, rps)],
                        out_hbm.at[c, pl.ds(s * rps, rps)])

    return k(hh_bf, src2, dst2, w2)


def _tc_matmul(a, w):
    n, din = a.shape
    dout = w.shape[1]
    bn = 2000

    def body(a_ref, w_ref, o_ref):
        o_ref[...] = jnp.dot(a_ref[...], w_ref[...],
                             preferred_element_type=jnp.float32)

    return pl.pallas_call(
        body,
        grid=(n // bn,),
        in_specs=[pl.BlockSpec((bn, din), lambda i: (i, 0)),
                  pl.BlockSpec((din, dout), lambda i: (0, 0))],
        out_specs=pl.BlockSpec((bn, dout), lambda i: (i, 0)),
        out_shape=jax.ShapeDtypeStruct((n, dout), jnp.float32),
    )(a, w)


def _tc_dinv_hh(degt, h):
    n, d = h.shape
    bn = 2000

    def body(deg_ref, h_ref, dinv_ref, hh_ref):
        deg = jnp.sum(deg_ref[...], axis=1, keepdims=True) + 1.0
        dinv = jnp.where(deg > 0, lax.rsqrt(deg), 0.0)
        dinv_ref[...] = dinv
        hh_ref[...] = dinv * h_ref[...]

    return pl.pallas_call(
        body,
        grid=(n // bn,),
        in_specs=[pl.BlockSpec((bn, NC), lambda i: (i, 0)),
                  pl.BlockSpec((bn, d), lambda i: (i, 0))],
        out_specs=[pl.BlockSpec((bn, 1), lambda i: (i, 0)),
                   pl.BlockSpec((bn, d), lambda i: (i, 0))],
        out_shape=[jax.ShapeDtypeStruct((n, 1), jnp.float32),
                   jax.ShapeDtypeStruct((n, d), jnp.float32)],
    )(degt, h)


def _tc_mid(parts, hh, dinv, b, w):
    n, d = hh.shape
    dout = w.shape[1]
    bn = 2000

    def body(p_ref, hh_ref, dinv_ref, b_ref, w_ref, o_ref):
        acc = p_ref[0] + p_ref[1] + hh_ref[...]
        act = jnp.maximum(dinv_ref[...] * acc + b_ref[...], 0.0)
        o_ref[...] = dinv_ref[...] * jnp.dot(
            act, w_ref[...], preferred_element_type=jnp.float32)

    return pl.pallas_call(
        body,
        grid=(n // bn,),
        in_specs=[pl.BlockSpec((NC, bn, d), lambda i: (0, i, 0)),
                  pl.BlockSpec((bn, d), lambda i: (i, 0)),
                  pl.BlockSpec((bn, 1), lambda i: (i, 0)),
                  pl.BlockSpec((1, d), lambda i: (0, 0)),
                  pl.BlockSpec((d, dout), lambda i: (0, 0))],
        out_specs=pl.BlockSpec((bn, dout), lambda i: (i, 0)),
        out_shape=jax.ShapeDtypeStruct((n, dout), jnp.float32),
    )(parts, hh, dinv, b, w)


def _tc_mid2(pa, pb, hh, dinv, b, w):
    n, d = hh.shape
    dh = d // 2
    dout = w.shape[1]
    bn = 2000

    def body(pa_ref, pb_ref, hh_ref, dinv_ref, b_ref, w_ref, o_ref):
        acc = jnp.concatenate(
            [pa_ref[0] + pa_ref[1], pb_ref[0] + pb_ref[1]], axis=1)
        acc = acc + hh_ref[...]
        act = jnp.maximum(dinv_ref[...] * acc + b_ref[...], 0.0)
        o_ref[...] = dinv_ref[...] * jnp.dot(
            act, w_ref[...], preferred_element_type=jnp.float32)

    return pl.pallas_call(
        body,
        grid=(n // bn,),
        in_specs=[pl.BlockSpec((NC, bn, dh), lambda i: (0, i, 0)),
                  pl.BlockSpec((NC, bn, dh), lambda i: (0, i, 0)),
                  pl.BlockSpec((bn, d), lambda i: (i, 0)),
                  pl.BlockSpec((bn, 1), lambda i: (i, 0)),
                  pl.BlockSpec((1, d), lambda i: (0, 0)),
                  pl.BlockSpec((d, dout), lambda i: (0, 0))],
        out_specs=pl.BlockSpec((bn, dout), lambda i: (i, 0)),
        out_shape=jax.ShapeDtypeStruct((n, dout), jnp.float32),
    )(pa, pb, hh, dinv, b, w)


def _tc_post(parts, hh, dinv, b):
    n, d = hh.shape
    bn = 2000

    def body(p_ref, hh_ref, dinv_ref, b_ref, o_ref):
        acc = p_ref[0] + p_ref[1] + hh_ref[...]
        o_ref[...] = dinv_ref[...] * acc + b_ref[...]

    return pl.pallas_call(
        body,
        grid=(n // bn,),
        in_specs=[pl.BlockSpec((NC, bn, d), lambda i: (0, i, 0)),
                  pl.BlockSpec((bn, d), lambda i: (i, 0)),
                  pl.BlockSpec((bn, 1), lambda i: (i, 0)),
                  pl.BlockSpec((1, d), lambda i: (0, 0))],
        out_specs=pl.BlockSpec((bn, d), lambda i: (i, 0)),
        out_shape=jax.ShapeDtypeStruct((n, d), jnp.float32),
    )(parts, hh, dinv, b)


def kernel(x, edge_index, edge_attr, W1, b1, W2, b2, W3, b3):
    n = x.shape[0]
    e = edge_attr.shape[0]
    ept = _round_up(-(-e // NW), G * 8)
    epad = ept * NW
    pad = epad - e
    npad = _round_up(n, 8 * NW)
    padrow = n if npad > n else n - 1
    srcp = jnp.concatenate([edge_index[0], jnp.zeros((pad,), jnp.int32)])
    dstp = jnp.concatenate([edge_index[1], jnp.full((pad,), padrow, jnp.int32)])
    wpad = jnp.concatenate([edge_attr, jnp.zeros((pad,), jnp.float32)])
    src2 = srcp.reshape(epad // G, G)
    dst2 = dstp.reshape(epad // G, G)
    w2 = wpad.reshape(epad // G, G)

    degp = _sc_degree(dst2, w2, n)
    h1 = _tc_matmul(x, W1)
    degt = jnp.transpose(degp)[:n]
    dinv, hh1 = _tc_dinv_hh(degt, h1)

    def prep(a):
        m, d = a.shape
        a = a.astype(jnp.bfloat16)
        a = a.reshape(m, d // 32, 2, 16).transpose(0, 1, 3, 2).reshape(m, d)
        return jnp.pad(a, ((0, npad - m), (0, 0)))

    dh = hh1.shape[1] // 2
    p1a = _sc_edge_pass(prep(hh1[:, :dh]), src2, dst2, w2)
    p1b = _sc_edge_pass(prep(hh1[:, dh:]), src2, dst2, w2)
    hh2 = _tc_mid2(p1a, p1b, hh1, dinv, b1.reshape(1, -1), W2)
    p2 = _sc_edge_pass(prep(hh2), src2, dst2, w2)
    hh3 = _tc_mid(p2, hh2, dinv, b2.reshape(1, -1), W3)
    p3 = _sc_edge_pass(prep(hh3), src2, dst2, w2)
    return _tc_post(p3, hh3, dinv, b3.reshape(1, -1))
